# baseline (speedup 1.0000x reference)
import functools

import jax
import jax.numpy as jnp
from jax import lax
from jax.experimental import pallas as pl
from jax.experimental.pallas import tpu as pltpu
from jax.experimental.pallas import tpu_sc as plsc

N = 10000
E = 320000
SIZE = 128
GATE = 3 * SIZE

NC = 2
NS = 16

CHUNK = 128
IB = 16
NB = 10
CPT = IB * NB
EPT = CPT * CHUNK
E_PAD = EPT * NS
N_PAD = 10112
RPT = N_PAD // NS


def _sc_aggregate(xin, xh, src2d, dst2d, zeros_feat, zeros_deg):
    mesh = plsc.VectorSubcoreMesh(core_axis_name="c", subcore_axis_name="s",
                                  num_cores=NC, num_subcores=NS)

    @functools.partial(
        pl.kernel,
        out_type=(
            jax.ShapeDtypeStruct((N_PAD, SIZE), jnp.float32),
            jax.ShapeDtypeStruct((N_PAD, SIZE), jnp.float32),
            jax.ShapeDtypeStruct((NS * N_PAD,), jnp.float32),
        ),
        mesh=mesh,
        scratch_types=[
            pltpu.VMEM((IB, CHUNK), jnp.int32),
            pltpu.VMEM((IB, CHUNK), jnp.int32),
            pltpu.VMEM((CHUNK, SIZE), jnp.float32),
            pltpu.VMEM((CHUNK, SIZE), jnp.float32),
            pltpu.VMEM((N_PAD,), jnp.float32),
            pltpu.VMEM_SHARED((N_PAD, SIZE), jnp.float32),
            pltpu.SemaphoreType.DMA,
            pltpu.SemaphoreType.DMA,
        ],
        compiler_params=pltpu.CompilerParams(needs_layout_passes=False),
    )
    def agg_kernel(xin_hbm, xh_hbm, src_hbm, dst_hbm, zf_hbm, zd_hbm,
                   out_in, out_h, out_deg,
                   sidx, didx, rows_a, rows_b, deg_v, acc_sh, sem_a, sem_b):
        cid = lax.axis_index("c")
        tid = lax.axis_index("s")
        r0 = tid * RPT

        pltpu.sync_copy(zf_hbm.at[pl.ds(r0, RPT)], acc_sh.at[pl.ds(r0, RPT)])
        pltpu.sync_copy(zd_hbm, deg_v)
        plsc.subcore_barrier()

        ones16 = jnp.full((16,), 1.0, jnp.float32)

        bufs = (rows_a, rows_b)
        sems = (sem_a, sem_b)

        def run_edges(table_hbm, with_deg):
            def block(b, _):
                c0 = tid * CPT + b * IB
                pltpu.sync_copy(src_hbm.at[pl.ds(c0, IB)], sidx)
                pltpu.sync_copy(dst_hbm.at[pl.ds(c0, IB)], didx)
                copies = [None] * IB
                copies[0] = pltpu.async_copy(
                    table_hbm.at[sidx.at[0]], bufs[0], sems[0])
                for i in range(IB):
                    if i + 1 < IB:
                        copies[i + 1] = pltpu.async_copy(
                            table_hbm.at[sidx.at[i + 1]],
                            bufs[(i + 1) % 2], sems[(i + 1) % 2])
                    copies[i].wait()
                    pltpu.sync_copy(bufs[i % 2], acc_sh.at[didx.at[i]],
                                    add=True)
                    if with_deg:
                        for j in range(CHUNK // 16):
                            idx16 = didx[i, pl.ds(j * 16, 16)]
                            plsc.addupdate_scatter(deg_v, [idx16], ones16)
                return 0
            lax.fori_loop(0, NB, block, 0)

        @pl.when(cid == 0)
        def _():
            run_edges(xin_hbm, True)

        @pl.when(cid == 1)
        def _():
            run_edges(xh_hbm, False)

        plsc.subcore_barrier()

        @pl.when(cid == 0)
        def _():
            pltpu.sync_copy(acc_sh.at[pl.ds(r0, RPT)],
                            out_in.at[pl.ds(r0, RPT)])
            pltpu.sync_copy(deg_v, out_deg.at[pl.ds(tid * N_PAD, N_PAD)])

        @pl.when(cid == 1)
        def _():
            pltpu.sync_copy(acc_sh.at[pl.ds(r0, RPT)],
                            out_h.at[pl.ds(r0, RPT)])

    return agg_kernel(xin, xh, src2d, dst2d, zeros_feat, zeros_deg)


ROWS_BLK = 1000


def _dense_body(xi_ref, xh_ref, ai_ref, ah_ref, deg_ref,
                wis_ref, win_ref, bi_ref, whs_ref, whn_ref, bh_ref,
                g_ref, b_ref, out_ref):
    deg = jnp.sum(deg_ref[...], axis=1, keepdims=True)
    recip = 1.0 / jnp.maximum(deg, 1.0)
    ai = ai_ref[...] * recip
    ah = ah_ref[...] * recip
    bf = jnp.bfloat16
    gi = (jnp.dot(xi_ref[...].astype(bf), wis_ref[...].astype(bf),
                  preferred_element_type=jnp.float32)
          + jnp.dot(ai.astype(bf), win_ref[...].astype(bf),
                    preferred_element_type=jnp.float32)
          + bi_ref[...])
    gh = (jnp.dot(xh_ref[...].astype(bf), whs_ref[...].astype(bf),
                  preferred_element_type=jnp.float32)
          + jnp.dot(ah.astype(bf), whn_ref[...].astype(bf),
                    preferred_element_type=jnp.float32)
          + bh_ref[...])
    i_r, i_i, i_n = gi[:, :SIZE], gi[:, SIZE:2 * SIZE], gi[:, 2 * SIZE:]
    h_r, h_i, h_n = gh[:, :SIZE], gh[:, SIZE:2 * SIZE], gh[:, 2 * SIZE:]
    resetgate = jax.nn.sigmoid(i_r + h_r)
    inputgate = jax.nn.sigmoid(i_i + h_i)
    newgate = jnp.tanh(i_n + resetgate * h_n)
    out = newgate + inputgate * (xh_ref[...] - newgate)
    mean = jnp.mean(out, axis=-1, keepdims=True)
    var = jnp.mean((out - mean) * (out - mean), axis=-1, keepdims=True)
    out = (out - mean) * jax.lax.rsqrt(var + 1e-5) * g_ref[...] + b_ref[...]
    out_ref[...] = out


def _dense(xi, xh, ai, ah, deg_parts, Wi_self, Wi_nbr, bi, Wh_self, Wh_nbr,
           bh, gamma, beta):
    grid = (N // ROWS_BLK,)
    row_spec = pl.BlockSpec((ROWS_BLK, SIZE), lambda i: (i, 0))
    deg_spec = pl.BlockSpec((ROWS_BLK, NS), lambda i: (i, 0))
    w_spec = pl.BlockSpec((SIZE, GATE), lambda i: (0, 0))
    b_spec = pl.BlockSpec((1, GATE), lambda i: (0, 0))
    gb_spec = pl.BlockSpec((1, SIZE), lambda i: (0, 0))
    return pl.pallas_call(
        _dense_body,
        grid=grid,
        in_specs=[row_spec, row_spec, row_spec, row_spec, deg_spec,
                  w_spec, w_spec, b_spec, w_spec, w_spec, b_spec,
                  gb_spec, gb_spec],
        out_specs=row_spec,
        out_shape=jax.ShapeDtypeStruct((N, SIZE), jnp.float32),
    )(xi, xh, ai, ah, deg_parts, Wi_self, Wi_nbr, bi, Wh_self, Wh_nbr, bh,
      gamma, beta)


def kernel(input, hidden, edge_index, Wi_self, Wi_nbr, bi, Wh_self, Wh_nbr,
           bh, gamma, beta):
    pad_e = E_PAD - E
    src = jnp.concatenate(
        [edge_index[0], jnp.zeros((pad_e,), jnp.int32)]).reshape(-1, CHUNK)
    dst = jnp.concatenate(
        [edge_index[1], jnp.full((pad_e,), N, jnp.int32)]).reshape(-1, CHUNK)
    zeros_feat = jnp.zeros((N_PAD, SIZE), jnp.float32)
    zeros_deg = jnp.zeros((N_PAD,), jnp.float32)

    sum_in, sum_h, deg_flat = _sc_aggregate(
        input, hidden, src, dst, zeros_feat, zeros_deg)

    deg_parts = deg_flat.reshape(NS, N_PAD).T[:N]

    return _dense(input, hidden, sum_in, sum_h, deg_parts,
                  Wi_self, Wi_nbr, bi.reshape(1, GATE),
                  Wh_self, Wh_nbr, bh.reshape(1, GATE),
                  gamma.reshape(1, SIZE), beta.reshape(1, SIZE))


# speedup vs baseline: 1.0060x; 1.0060x over previous
import functools

import jax
import jax.numpy as jnp
from jax import lax
from jax.experimental import pallas as pl
from jax.experimental.pallas import tpu as pltpu
from jax.experimental.pallas import tpu_sc as plsc

N = 10000
E = 320000
SIZE = 128
GATE = 3 * SIZE

NC = 2
NS = 16

CHUNK = 128
IB = 16
NB = 10
CPT = IB * NB
EPT = CPT * CHUNK
E_PAD = EPT * NS
N_PAD = 10112
RPT = N_PAD // NS


def _sc_aggregate(xin, xh, src2d, dst2d, zeros_feat, zeros_deg):
    mesh = plsc.VectorSubcoreMesh(core_axis_name="c", subcore_axis_name="s",
                                  num_cores=NC, num_subcores=NS)

    @functools.partial(
        pl.kernel,
        out_type=(
            jax.ShapeDtypeStruct((N_PAD, SIZE), jnp.float32),
            jax.ShapeDtypeStruct((N_PAD, SIZE), jnp.float32),
            jax.ShapeDtypeStruct((NS * N_PAD,), jnp.float32),
        ),
        mesh=mesh,
        scratch_types=[
            pltpu.VMEM((IB, CHUNK), jnp.int32),
            pltpu.VMEM((IB, CHUNK), jnp.int32),
            pltpu.VMEM((CHUNK, SIZE), jnp.float32),
            pltpu.VMEM((CHUNK, SIZE), jnp.float32),
            pltpu.VMEM((N_PAD,), jnp.float32),
            pltpu.VMEM_SHARED((N_PAD, SIZE), jnp.float32),
            pltpu.SemaphoreType.DMA,
            pltpu.SemaphoreType.DMA,
        ],
        compiler_params=pltpu.CompilerParams(needs_layout_passes=False),
    )
    def agg_kernel(xin_hbm, xh_hbm, src_hbm, dst_hbm, zf_hbm, zd_hbm,
                   out_in, out_h, out_deg,
                   sidx, didx, rows_a, rows_b, deg_v, acc_sh, sem_a, sem_b):
        cid = lax.axis_index("c")
        tid = lax.axis_index("s")
        r0 = tid * RPT

        pltpu.sync_copy(zf_hbm.at[pl.ds(r0, RPT)], acc_sh.at[pl.ds(r0, RPT)])
        pltpu.sync_copy(zd_hbm, deg_v)
        plsc.subcore_barrier()

        ones16 = jnp.full((16,), 1.0, jnp.float32)

        bufs = (rows_a, rows_b)
        sems = (sem_a, sem_b)

        def run_edges(table_hbm, with_deg):
            def block(b, _):
                c0 = tid * CPT + b * IB
                pltpu.sync_copy(src_hbm.at[pl.ds(c0, IB)], sidx)
                pltpu.sync_copy(dst_hbm.at[pl.ds(c0, IB)], didx)
                copies = [None] * IB
                copies[0] = pltpu.async_copy(
                    table_hbm.at[sidx.at[0]], bufs[0], sems[0])
                for i in range(IB):
                    if i + 1 < IB:
                        copies[i + 1] = pltpu.async_copy(
                            table_hbm.at[sidx.at[i + 1]],
                            bufs[(i + 1) % 2], sems[(i + 1) % 2])
                    copies[i].wait()
                    pltpu.sync_copy(bufs[i % 2], acc_sh.at[didx.at[i]],
                                    add=True)
                    if with_deg:
                        for j in range(CHUNK // 16):
                            idx16 = didx[i, pl.ds(j * 16, 16)]
                            plsc.addupdate_scatter(deg_v, [idx16], ones16)
                return 0
            lax.fori_loop(0, NB, block, 0)

        @pl.when(cid == 0)
        def _():
            run_edges(xin_hbm, True)

        @pl.when(cid == 1)
        def _():
            run_edges(xh_hbm, False)

        plsc.subcore_barrier()

        @pl.when(cid == 0)
        def _():
            pltpu.sync_copy(acc_sh.at[pl.ds(r0, RPT)],
                            out_in.at[pl.ds(r0, RPT)])
            pltpu.sync_copy(deg_v, out_deg.at[pl.ds(tid * N_PAD, N_PAD)])

        @pl.when(cid == 1)
        def _():
            pltpu.sync_copy(acc_sh.at[pl.ds(r0, RPT)],
                            out_h.at[pl.ds(r0, RPT)])

    return agg_kernel(xin, xh, src2d, dst2d, zeros_feat, zeros_deg)


ROWS_BLK = 1000


def _dense_body(xi_ref, xh_ref, ai_ref, ah_ref, deg_ref,
                wis_ref, win_ref, bi_ref, whs_ref, whn_ref, bh_ref,
                g_ref, b_ref, out_ref):
    deg = jnp.sum(deg_ref[...], axis=1, keepdims=True)
    recip = 1.0 / jnp.maximum(deg, 1.0)
    ai = ai_ref[...] * recip
    ah = ah_ref[...] * recip
    gi = (jnp.dot(xi_ref[...], wis_ref[...], preferred_element_type=jnp.float32)
          + jnp.dot(ai, win_ref[...], preferred_element_type=jnp.float32)
          + bi_ref[...])
    gh = (jnp.dot(xh_ref[...], whs_ref[...], preferred_element_type=jnp.float32)
          + jnp.dot(ah, whn_ref[...], preferred_element_type=jnp.float32)
          + bh_ref[...])
    i_r, i_i, i_n = gi[:, :SIZE], gi[:, SIZE:2 * SIZE], gi[:, 2 * SIZE:]
    h_r, h_i, h_n = gh[:, :SIZE], gh[:, SIZE:2 * SIZE], gh[:, 2 * SIZE:]
    resetgate = jax.nn.sigmoid(i_r + h_r)
    inputgate = jax.nn.sigmoid(i_i + h_i)
    newgate = jnp.tanh(i_n + resetgate * h_n)
    out = newgate + inputgate * (xh_ref[...] - newgate)
    mean = jnp.mean(out, axis=-1, keepdims=True)
    var = jnp.mean((out - mean) * (out - mean), axis=-1, keepdims=True)
    out = (out - mean) * jax.lax.rsqrt(var + 1e-5) * g_ref[...] + b_ref[...]
    out_ref[...] = out


def _dense(xi, xh, ai, ah, deg_parts, Wi_self, Wi_nbr, bi, Wh_self, Wh_nbr,
           bh, gamma, beta):
    grid = (N // ROWS_BLK,)
    row_spec = pl.BlockSpec((ROWS_BLK, SIZE), lambda i: (i, 0))
    deg_spec = pl.BlockSpec((ROWS_BLK, NS), lambda i: (i, 0))
    w_spec = pl.BlockSpec((SIZE, GATE), lambda i: (0, 0))
    b_spec = pl.BlockSpec((1, GATE), lambda i: (0, 0))
    gb_spec = pl.BlockSpec((1, SIZE), lambda i: (0, 0))
    return pl.pallas_call(
        _dense_body,
        grid=grid,
        in_specs=[row_spec, row_spec, row_spec, row_spec, deg_spec,
                  w_spec, w_spec, b_spec, w_spec, w_spec, b_spec,
                  gb_spec, gb_spec],
        out_specs=row_spec,
        out_shape=jax.ShapeDtypeStruct((N, SIZE), jnp.float32),
    )(xi, xh, ai, ah, deg_parts, Wi_self, Wi_nbr, bi, Wh_self, Wh_nbr, bh,
      gamma, beta)


def kernel(input, hidden, edge_index, Wi_self, Wi_nbr, bi, Wh_self, Wh_nbr,
           bh, gamma, beta):
    pad_e = E_PAD - E
    src = jnp.concatenate(
        [edge_index[0], jnp.zeros((pad_e,), jnp.int32)]).reshape(-1, CHUNK)
    dst = jnp.concatenate(
        [edge_index[1], jnp.full((pad_e,), N, jnp.int32)]).reshape(-1, CHUNK)
    zeros_feat = jnp.zeros((N_PAD, SIZE), jnp.float32)
    zeros_deg = jnp.zeros((N_PAD,), jnp.float32)

    sum_in, sum_h, deg_flat = _sc_aggregate(
        input, hidden, src, dst, zeros_feat, zeros_deg)

    deg_parts = deg_flat.reshape(NS, N_PAD).T[:N]

    return _dense(input, hidden, sum_in, sum_h, deg_parts,
                  Wi_self, Wi_nbr, bi.reshape(1, GATE),
                  Wh_self, Wh_nbr, bh.reshape(1, GATE),
                  gamma.reshape(1, SIZE), beta.reshape(1, SIZE))
